# Initial kernel scaffold; baseline (speedup 1.0000x reference)
import jax
import jax.numpy as jnp
from jax.experimental import pallas as pl


def kernel(x, edge_index, edge_attr, batch, W1, b1, W2, b2):
    raise NotImplementedError("write your pallas kernel here")


# speedup vs baseline: 4.2162x; 4.2162x over previous
import functools

import jax
import jax.numpy as jnp
from jax import lax
from jax.experimental import pallas as pl
from jax.experimental.pallas import tpu as pltpu
from jax.experimental.pallas import tpu_sc as plsc

N_NODES = 10000
N_EDGES = 320000
D = 128
N_GRAPHS = 64

NC = 2
NS = 16
NW = NC * NS
EW = N_EDGES // NW
CH = 128
NFULL = EW // CH
TAIL = EW - NFULL * CH
RPT = 632
RPAD = RPT * NS

BN = 400
NB = N_NODES // BN


def _edge_aggr_body(x_hbm, src_hbm, dst_hbm, ea_hbm, out_hbm,
                    xrows, ebuf, srcbuf, dstbuf, src_t, dst_t,
                    aggr_sp, sem):
    cid = lax.axis_index("c")
    sid = lax.axis_index("s")
    wid = cid * NS + sid

    def zrow(r, carry):
        for j in range(8):
            xrows[r, pl.ds(j * 16, 16)] = jnp.zeros((16,), jnp.float32)
        return carry
    lax.fori_loop(0, CH, zrow, 0)
    for k in range(4):
        pltpu.sync_copy(xrows,
                        aggr_sp.at[pl.ds(sid * RPT + k * CH, CH)])
    pltpu.sync_copy(xrows.at[pl.ds(0, RPT - 4 * CH)],
                    aggr_sp.at[pl.ds(sid * RPT + 4 * CH, RPT - 4 * CH)])
    plsc.subcore_barrier()

    ebase = wid * EW

    def compute_rows(n):
        def crow(r, carry):
            for j in range(8):
                sl = pl.ds(j * 16, 16)
                xrows[r, sl] = jnp.maximum(xrows[r, sl] + ebuf[r, sl], 0.0)
            return carry
        lax.fori_loop(0, n, crow, 0)

    def chunk(c, carry):
        eb = ebase + c * CH
        pltpu.sync_copy(src_hbm.at[pl.ds(eb, CH)], srcbuf)
        pltpu.sync_copy(dst_hbm.at[pl.ds(eb, CH)], dstbuf)
        gather = pltpu.async_copy(x_hbm.at[srcbuf], xrows, sem)
        pltpu.sync_copy(ea_hbm.at[pl.ds(eb, CH)], ebuf)
        gather.wait()
        compute_rows(CH)
        pltpu.sync_copy(xrows, aggr_sp.at[dstbuf], add=True)
        return carry
    lax.fori_loop(0, NFULL, chunk, 0)

    eb = ebase + NFULL * CH
    pltpu.sync_copy(src_hbm.at[pl.ds(eb, TAIL)], src_t)
    pltpu.sync_copy(dst_hbm.at[pl.ds(eb, TAIL)], dst_t)
    gather = pltpu.async_copy(x_hbm.at[src_t], xrows.at[pl.ds(0, TAIL)], sem)
    pltpu.sync_copy(ea_hbm.at[pl.ds(eb, TAIL)], ebuf.at[pl.ds(0, TAIL)])
    gather.wait()
    compute_rows(TAIL)
    pltpu.sync_copy(xrows.at[pl.ds(0, TAIL)], aggr_sp.at[dst_t], add=True)

    plsc.subcore_barrier()
    pltpu.sync_copy(aggr_sp.at[pl.ds(sid * RPT, RPT)],
                    out_hbm.at[cid, pl.ds(sid * RPT, RPT)])


@functools.lru_cache(maxsize=None)
def _edge_aggr_call():
    return functools.partial(
        pl.kernel,
        out_type=jax.ShapeDtypeStruct((NC, RPAD, D), jnp.float32),
        mesh=plsc.VectorSubcoreMesh(
            core_axis_name="c", subcore_axis_name="s", num_cores=NC),
        scratch_types=[
            pltpu.VMEM((CH, D), jnp.float32),
            pltpu.VMEM((CH, D), jnp.float32),
            pltpu.VMEM((CH,), jnp.int32),
            pltpu.VMEM((CH,), jnp.int32),
            pltpu.VMEM((TAIL,), jnp.int32),
            pltpu.VMEM((TAIL,), jnp.int32),
            pltpu.VMEM_SHARED((RPAD, D), jnp.float32),
            pltpu.SemaphoreType.DMA,
        ],
    )(_edge_aggr_body)


def _edge_aggr(x, src, dst, ea):
    return _edge_aggr_call()(x, src, dst, ea)


def _mlp_kernel(x_ref, a0_ref, a1_ref, w1_ref, b1_ref, w2_ref, b2_ref, o_ref):
    t = x_ref[...] + a0_ref[...] + a1_ref[...]
    h = jnp.maximum(
        jnp.dot(t, w1_ref[...], preferred_element_type=jnp.float32)
        + b1_ref[...], 0.0)
    h = jnp.dot(h, w2_ref[...], preferred_element_type=jnp.float32) + b2_ref[...]
    o_ref[...] = jnp.maximum(h, 0.0)


def _mlp(x, a0, a1, w1, b1, w2, b2):
    return pl.pallas_call(
        _mlp_kernel,
        grid=(NB,),
        in_specs=[
            pl.BlockSpec((BN, D), lambda i: (i, 0)),
            pl.BlockSpec((BN, D), lambda i: (i, 0)),
            pl.BlockSpec((BN, D), lambda i: (i, 0)),
            pl.BlockSpec((D, D), lambda i: (0, 0)),
            pl.BlockSpec((1, D), lambda i: (0, 0)),
            pl.BlockSpec((D, D), lambda i: (0, 0)),
            pl.BlockSpec((1, D), lambda i: (0, 0)),
        ],
        out_specs=pl.BlockSpec((BN, D), lambda i: (i, 0)),
        out_shape=jax.ShapeDtypeStruct((N_NODES, D), jnp.float32),
    )(x, a0, a1, w1, b1, w2, b2)


def _mlp_pool_kernel(x_ref, a0_ref, a1_ref, w1_ref, b1_ref, w2_ref, b2_ref,
                     bat_ref, o_ref, sums, counts):
    i = pl.program_id(0)
    t = x_ref[...] + a0_ref[...] + a1_ref[...]
    h = jnp.maximum(
        jnp.dot(t, w1_ref[...], preferred_element_type=jnp.float32)
        + b1_ref[...], 0.0)
    h = jnp.dot(h, w2_ref[...], preferred_element_type=jnp.float32) + b2_ref[...]
    h = jnp.maximum(h, 0.0)

    bb = bat_ref[...].reshape(1, BN)
    onehot = (lax.broadcasted_iota(jnp.int32, (N_GRAPHS, BN), 0)
              == jnp.broadcast_to(bb, (N_GRAPHS, BN))).astype(jnp.float32)
    part = jnp.dot(onehot, h, preferred_element_type=jnp.float32)
    cnt = jnp.broadcast_to(jnp.sum(onehot, axis=1, keepdims=True),
                           (N_GRAPHS, D))

    @pl.when(i == 0)
    def _():
        sums[...] = part
        counts[...] = cnt

    @pl.when(i > 0)
    def _():
        sums[...] = sums[...] + part
        counts[...] = counts[...] + cnt

    @pl.when(i == NB - 1)
    def _():
        o_ref[...] = sums[...] / jnp.maximum(counts[...], 1.0)


def _mlp_pool(x, a0, a1, w1, b1, w2, b2, bat3):
    return pl.pallas_call(
        _mlp_pool_kernel,
        grid=(NB,),
        in_specs=[
            pl.BlockSpec((BN, D), lambda i: (i, 0)),
            pl.BlockSpec((BN, D), lambda i: (i, 0)),
            pl.BlockSpec((BN, D), lambda i: (i, 0)),
            pl.BlockSpec((D, D), lambda i: (0, 0)),
            pl.BlockSpec((1, D), lambda i: (0, 0)),
            pl.BlockSpec((D, D), lambda i: (0, 0)),
            pl.BlockSpec((1, D), lambda i: (0, 0)),
            pl.BlockSpec((1, 1, BN), lambda i: (i, 0, 0)),
        ],
        out_specs=pl.BlockSpec((N_GRAPHS, D), lambda i: (0, 0)),
        out_shape=jax.ShapeDtypeStruct((N_GRAPHS, D), jnp.float32),
        scratch_shapes=[
            pltpu.VMEM((N_GRAPHS, D), jnp.float32),
            pltpu.VMEM((N_GRAPHS, D), jnp.float32),
        ],
    )(x, a0, a1, w1, b1, w2, b2, bat3)


@jax.jit
def kernel(x, edge_index, edge_attr, batch, W1, b1, W2, b2):
    src = edge_index[0].astype(jnp.int32)
    dst = edge_index[1].astype(jnp.int32)
    b1r = b1.reshape(1, D)
    b2r = b2.reshape(1, D)
    bat3 = batch.astype(jnp.int32).reshape(NB, 1, BN)

    p = _edge_aggr(x, src, dst, edge_attr)
    h1 = _mlp(x, p[0, :N_NODES], p[1, :N_NODES], W1, b1r, W2, b2r)
    p2 = _edge_aggr(h1, src, dst, edge_attr)
    return _mlp_pool(h1, p2[0, :N_NODES], p2[1, :N_NODES],
                     W1, b1r, W2, b2r, bat3)


# speedup vs baseline: 7.4046x; 1.7562x over previous
import functools

import jax
import jax.numpy as jnp
from jax import lax
from jax.experimental import pallas as pl
from jax.experimental.pallas import tpu as pltpu
from jax.experimental.pallas import tpu_sc as plsc

N_NODES = 10000
N_EDGES = 320000
D = 128
N_GRAPHS = 64

NC = 2
NS = 16
NW = NC * NS
EW = N_EDGES // NW
CH = 64
NFULL = EW // CH
TAIL = EW - NFULL * CH
RPT = 632
RPAD = RPT * NS

BN = 400
NB = N_NODES // BN


def _edge_aggr_body(x_hbm, src_hbm, dst_hbm, ea_hbm, out_hbm,
                    xrows0, xrows1, ebuf0, ebuf1, dstb0, dstb1,
                    src_v, src_t, dst_t, aggr_sp, sem0, sem1):
    cid = lax.axis_index("c")
    sid = lax.axis_index("s")
    wid = cid * NS + sid
    ebase = wid * EW

    xrows = (xrows0, xrows1)
    ebuf = (ebuf0, ebuf1)
    dstb = (dstb0, dstb1)
    sem = (sem0, sem1)

    pltpu.sync_copy(src_hbm.at[pl.ds(ebase, NFULL * CH)], src_v)

    def prefetch(k, s):
        eb = ebase + k * CH
        pltpu.async_copy(dst_hbm.at[pl.ds(eb, CH)], dstb[s], sem[s])
        pltpu.async_copy(x_hbm.at[src_v.at[pl.ds(k * CH, CH)]],
                         xrows[s], sem[s])
        pltpu.async_copy(ea_hbm.at[pl.ds(eb, CH)], ebuf[s], sem[s])

    def wait_slot(k, s):
        eb = ebase + k * CH
        pltpu.make_async_copy(dst_hbm.at[pl.ds(eb, CH)], dstb[s],
                              sem[s]).wait()
        pltpu.make_async_copy(x_hbm.at[src_v.at[pl.ds(k * CH, CH)]],
                              xrows[s], sem[s]).wait()
        pltpu.make_async_copy(ea_hbm.at[pl.ds(eb, CH)], ebuf[s],
                              sem[s]).wait()

    def compute_rows(s, n):
        def crow(r, carry):
            for j in range(8):
                sl = pl.ds(j * 16, 16)
                xrows[s][r, sl] = jnp.maximum(
                    xrows[s][r, sl] + ebuf[s][r, sl], 0.0)
            return carry
        lax.fori_loop(0, n, crow, 0)

    prefetch(0, 0)

    def zrow(r, carry):
        for j in range(8):
            ebuf1[r, pl.ds(j * 16, 16)] = jnp.zeros((16,), jnp.float32)
        return carry
    lax.fori_loop(0, CH, zrow, 0)
    nz = RPT // CH
    rem = RPT - nz * CH
    for k in range(nz):
        pltpu.sync_copy(ebuf1,
                        aggr_sp.at[pl.ds(sid * RPT + k * CH, CH)])
    pltpu.sync_copy(ebuf1.at[pl.ds(0, rem)],
                    aggr_sp.at[pl.ds(sid * RPT + nz * CH, rem)])
    plsc.subcore_barrier()

    def step(k, s):
        prefetch(k + 1, s ^ 1)
        wait_slot(k, s)
        compute_rows(s, CH)
        pltpu.sync_copy(xrows[s], aggr_sp.at[dstb[s]], add=True)

    def pair(c, carry):
        step(2 * c, 0)
        step(2 * c + 1, 1)
        return carry
    lax.fori_loop(0, NFULL // 2 - 1, pair, 0)

    k0 = NFULL - 2
    step(k0, 0)
    eb = ebase + NFULL * CH
    pltpu.async_copy(src_hbm.at[pl.ds(eb, TAIL)], src_t, sem0)
    pltpu.async_copy(dst_hbm.at[pl.ds(eb, TAIL)], dst_t, sem0)
    wait_slot(k0 + 1, 1)
    compute_rows(1, CH)
    pltpu.sync_copy(xrows[1], aggr_sp.at[dstb[1]], add=True)
    pltpu.make_async_copy(src_hbm.at[pl.ds(eb, TAIL)], src_t, sem0).wait()
    pltpu.make_async_copy(dst_hbm.at[pl.ds(eb, TAIL)], dst_t, sem0).wait()
    gather = pltpu.async_copy(x_hbm.at[src_t], xrows0.at[pl.ds(0, TAIL)],
                              sem0)
    pltpu.sync_copy(ea_hbm.at[pl.ds(eb, TAIL)], ebuf0.at[pl.ds(0, TAIL)])
    gather.wait()
    compute_rows(0, TAIL)
    pltpu.sync_copy(xrows0.at[pl.ds(0, TAIL)], aggr_sp.at[dst_t], add=True)

    plsc.subcore_barrier()
    pltpu.sync_copy(aggr_sp.at[pl.ds(sid * RPT, RPT)],
                    out_hbm.at[cid, pl.ds(sid * RPT, RPT)])


@functools.lru_cache(maxsize=None)
def _edge_aggr_call():
    return functools.partial(
        pl.kernel,
        out_type=jax.ShapeDtypeStruct((NC, RPAD, D), jnp.float32),
        mesh=plsc.VectorSubcoreMesh(
            core_axis_name="c", subcore_axis_name="s", num_cores=NC),
        scratch_types=[
            pltpu.VMEM((CH, D), jnp.float32),
            pltpu.VMEM((CH, D), jnp.float32),
            pltpu.VMEM((CH, D), jnp.float32),
            pltpu.VMEM((CH, D), jnp.float32),
            pltpu.VMEM((CH,), jnp.int32),
            pltpu.VMEM((CH,), jnp.int32),
            pltpu.VMEM((NFULL * CH,), jnp.int32),
            pltpu.VMEM((TAIL,), jnp.int32),
            pltpu.VMEM((TAIL,), jnp.int32),
            pltpu.VMEM_SHARED((RPAD, D), jnp.float32),
            pltpu.SemaphoreType.DMA,
            pltpu.SemaphoreType.DMA,
        ],
    )(_edge_aggr_body)


def _edge_aggr(x, src, dst, ea):
    return _edge_aggr_call()(x, src, dst, ea)


def _mlp_kernel(x_ref, a0_ref, a1_ref, w1_ref, b1_ref, w2_ref, b2_ref, o_ref):
    t = x_ref[...] + a0_ref[...] + a1_ref[...]
    h = jnp.maximum(
        jnp.dot(t, w1_ref[...], preferred_element_type=jnp.float32)
        + b1_ref[...], 0.0)
    h = jnp.dot(h, w2_ref[...], preferred_element_type=jnp.float32) + b2_ref[...]
    o_ref[...] = jnp.maximum(h, 0.0)


def _mlp(x, a0, a1, w1, b1, w2, b2):
    return pl.pallas_call(
        _mlp_kernel,
        grid=(NB,),
        in_specs=[
            pl.BlockSpec((BN, D), lambda i: (i, 0)),
            pl.BlockSpec((BN, D), lambda i: (i, 0)),
            pl.BlockSpec((BN, D), lambda i: (i, 0)),
            pl.BlockSpec((D, D), lambda i: (0, 0)),
            pl.BlockSpec((1, D), lambda i: (0, 0)),
            pl.BlockSpec((D, D), lambda i: (0, 0)),
            pl.BlockSpec((1, D), lambda i: (0, 0)),
        ],
        out_specs=pl.BlockSpec((BN, D), lambda i: (i, 0)),
        out_shape=jax.ShapeDtypeStruct((N_NODES, D), jnp.float32),
    )(x, a0, a1, w1, b1, w2, b2)


def _mlp_pool_kernel(x_ref, a0_ref, a1_ref, w1_ref, b1_ref, w2_ref, b2_ref,
                     bat_ref, o_ref, sums, counts):
    i = pl.program_id(0)
    t = x_ref[...] + a0_ref[...] + a1_ref[...]
    h = jnp.maximum(
        jnp.dot(t, w1_ref[...], preferred_element_type=jnp.float32)
        + b1_ref[...], 0.0)
    h = jnp.dot(h, w2_ref[...], preferred_element_type=jnp.float32) + b2_ref[...]
    h = jnp.maximum(h, 0.0)

    bb = bat_ref[...].reshape(1, BN)
    onehot = (lax.broadcasted_iota(jnp.int32, (N_GRAPHS, BN), 0)
              == jnp.broadcast_to(bb, (N_GRAPHS, BN))).astype(jnp.float32)
    part = jnp.dot(onehot, h, preferred_element_type=jnp.float32)
    cnt = jnp.broadcast_to(jnp.sum(onehot, axis=1, keepdims=True),
                           (N_GRAPHS, D))

    @pl.when(i == 0)
    def _():
        sums[...] = part
        counts[...] = cnt

    @pl.when(i > 0)
    def _():
        sums[...] = sums[...] + part
        counts[...] = counts[...] + cnt

    @pl.when(i == NB - 1)
    def _():
        o_ref[...] = sums[...] / jnp.maximum(counts[...], 1.0)


def _mlp_pool(x, a0, a1, w1, b1, w2, b2, bat3):
    return pl.pallas_call(
        _mlp_pool_kernel,
        grid=(NB,),
        in_specs=[
            pl.BlockSpec((BN, D), lambda i: (i, 0)),
            pl.BlockSpec((BN, D), lambda i: (i, 0)),
            pl.BlockSpec((BN, D), lambda i: (i, 0)),
            pl.BlockSpec((D, D), lambda i: (0, 0)),
            pl.BlockSpec((1, D), lambda i: (0, 0)),
            pl.BlockSpec((D, D), lambda i: (0, 0)),
            pl.BlockSpec((1, D), lambda i: (0, 0)),
            pl.BlockSpec((1, 1, BN), lambda i: (i, 0, 0)),
        ],
        out_specs=pl.BlockSpec((N_GRAPHS, D), lambda i: (0, 0)),
        out_shape=jax.ShapeDtypeStruct((N_GRAPHS, D), jnp.float32),
        scratch_shapes=[
            pltpu.VMEM((N_GRAPHS, D), jnp.float32),
            pltpu.VMEM((N_GRAPHS, D), jnp.float32),
        ],
    )(x, a0, a1, w1, b1, w2, b2, bat3)


@jax.jit
def kernel(x, edge_index, edge_attr, batch, W1, b1, W2, b2):
    src = edge_index[0].astype(jnp.int32)
    dst = edge_index[1].astype(jnp.int32)
    b1r = b1.reshape(1, D)
    b2r = b2.reshape(1, D)
    bat3 = batch.astype(jnp.int32).reshape(NB, 1, BN)

    p = _edge_aggr(x, src, dst, edge_attr)
    h1 = _mlp(x, p[0, :N_NODES], p[1, :N_NODES], W1, b1r, W2, b2r)
    p2 = _edge_aggr(h1, src, dst, edge_attr)
    return _mlp_pool(h1, p2[0, :N_NODES], p2[1, :N_NODES],
                     W1, b1r, W2, b2r, bat3)
